# baseline (speedup 1.0000x reference)
import jax
import jax.numpy as jnp
from jax import lax
from jax.experimental import pallas as pl
from jax.experimental.pallas import tpu as pltpu
from jax.experimental.pallas import tpu_sc as plsc

_B = 4096
_L = 200
_D = 64
_V = 1000000
_N = _B * _L
_NW = 32
_PER_W = _N // _NW
_CHUNK = 512
_CPW = _PER_W // _CHUNK
_JBLK = _CHUNK // 128
_CPL = _B // _CHUNK
_OTP = 129
_VBF = _V // 128
_TAIL0 = _VBF * 128
_BLKP = 137
_KPW_A = (_VBF + _NW - 1) // _NW


def _body_a(tt_hbm, tailp_hbm, out_hbm, blk0, blk1, stg0, stg1,
            dsem0, dsem1, wsem0, wsem1):
    cid = lax.axis_index("c")
    sid = lax.axis_index("s")
    wid = sid * 2 + cid

    lane16 = lax.iota(jnp.int32, 16)
    dvec = [g * 16 + lane16 for g in range(_D // 16)]

    blks = (blk0, blk1)
    stgs = (stg0, stg1)
    dsems = (dsem0, dsem1)
    wsems = (wsem0, wsem1)

    def in_copy(vb, p):
        return pltpu.make_async_copy(
            tt_hbm.at[pl.ds(0, _D), pl.ds(vb * 128, 128)],
            blks[p].at[:, pl.ds(0, 128)],
            dsems[p],
        )

    def out_copy(vb, p):
        return pltpu.make_async_copy(
            stgs[p], out_hbm.at[pl.ds(vb * 8192, 8192)], wsems[p]
        )

    def vb_of(k):
        return wid + k * _NW

    in_copy(vb_of(0), 0).start()

    def two_steps(h, carry):
        for p in range(2):
            k = 2 * h + p
            vb = vb_of(k)
            nvb = vb_of(k + 1)

            @pl.when(nvb < _VBF)
            def _start_next():
                in_copy(nvb, (p + 1) % 2).start()

            @pl.when(vb < _VBF)
            def _do_block():
                in_copy(vb, p).wait()

                @pl.when(k >= 2)
                def _wait_prev_write():
                    out_copy(vb, p).wait()

                blk = blks[p]
                stg = stgs[p]

                @plsc.parallel_loop(0, 128, step=4)
                def _sh(lane0):
                    for u in range(4):
                        lane = lane0 + u
                        lane_b = jnp.full((16,), lane, jnp.int32)
                        for g in range(_D // 16):
                            vals = plsc.load_gather(blk, [dvec[g], lane_b])
                            stg[pl.ds(lane * _D + g * 16, 16)] = vals

                out_copy(vb, p).start()
        return carry

    lax.fori_loop(0, (_KPW_A + 1) // 2, two_steps, 0)

    @pl.when(vb_of(_KPW_A - 1) < _VBF)
    def _drain1():
        out_copy(0, (_KPW_A - 1) % 2).wait()

    @pl.when((_KPW_A >= 2) & (vb_of(_KPW_A - 2) < _VBF))
    def _drain0():
        out_copy(0, (_KPW_A - 2) % 2).wait()

    @pl.when(wid == 0)
    def _tail():
        pltpu.sync_copy(tailp_hbm, blk0.at[:, pl.ds(0, 128)])
        for r in range(_V - _TAIL0):
            for g in range(_D // 16):
                stg0[pl.ds(r * _D + g * 16, 16)] = blk0[r, pl.ds(g * 16, 16)]
        pltpu.sync_copy(
            stg0.at[pl.ds(0, (_V - _TAIL0) * _D)],
            out_hbm.at[pl.ds(_TAIL0 * _D, (_V - _TAIL0) * _D)],
        )


def _body_b(idx_hbm, table_hbm, o5_hbm, idx_v, rows0, rows1, ot0, ot1,
            gsem0, gsem1, wsem0, wsem1):
    wid = lax.axis_index("s") * 2 + lax.axis_index("c")
    base_chunk = wid * _CPW

    pltpu.sync_copy(idx_hbm.at[pl.ds(wid * _PER_W, _PER_W)], idx_v)

    lane16 = lax.iota(jnp.int32, 16)
    tsg = []
    for g in range(_D // 16):
        d = g * 16 + lane16
        tsg.append((d >> 3, d & 7))

    rows = (rows0, rows1)
    gsems = (gsem0, gsem1)
    ots = (ot0, ot1)
    wsems = (wsem0, wsem1)

    def gather_copy(ci, p):
        return pltpu.make_async_copy(
            table_hbm.at[idx_v.at[pl.ds(ci * _CHUNK, _CHUNK)]],
            rows[p],
            gsems[p],
        )

    def write_copy(l, bj, q):
        return pltpu.make_async_copy(
            ots[q].at[:, :, pl.ds(0, 128)],
            o5_hbm.at[l, :, bj],
            wsems[q],
        )

    def process_chunk(ci, p):
        c = base_chunk + ci
        l = c // _CPL
        bblk0 = (c % _CPL) * _JBLK
        for j in range(_JBLK):
            q = j % 2
            m = ci * _JBLK + j

            @pl.when(m >= 2)
            def _wait_prev():
                write_copy(l, bblk0 + j, q).wait()

            @plsc.parallel_loop(0, 128, step=8)
            def _rowblk(rr0):
                for u in range(8):
                    rr = rr0 + u
                    lane_b = jnp.full((16,), rr, jnp.int32)
                    r = j * 128 + rr
                    for g in range(_D // 16):
                        vals = rows[p][r, pl.ds(g * 16, 16)]
                        plsc.store_scatter(
                            ots[q], [tsg[g][0], tsg[g][1], lane_b], vals
                        )

            write_copy(l, bblk0 + j, q).start()
        return l

    gather_copy(0, 0).start()

    def two_chunks(h, carry):
        for p in range(2):
            ci = 2 * h + p
            nci = ci + 1

            @pl.when(nci < _CPW)
            def _start_next():
                gather_copy(nci, (p + 1) % 2).start()

            gather_copy(ci, p).wait()
            process_chunk(ci, p)
        return carry

    lax.fori_loop(0, _CPW // 2, two_chunks, 0)

    write_copy(_L - 1, _B // 128 - 1, 0).wait()
    write_copy(_L - 1, _B // 128 - 1, 1).wait()


def kernel(x, table):
    idx = x.T.reshape(_N)
    tailp = jnp.pad(table[_TAIL0:, :], ((0, 0), (0, 128 - _D)))
    mesh = plsc.VectorSubcoreMesh(core_axis_name="c", subcore_axis_name="s")

    k_a = pl.kernel(
        _body_a,
        out_type=jax.ShapeDtypeStruct((_V * _D,), jnp.float32),
        mesh=mesh,
        scratch_types=[
            pltpu.VMEM((_D, _BLKP), jnp.float32),
            pltpu.VMEM((_D, _BLKP), jnp.float32),
            pltpu.VMEM((128 * _D,), jnp.float32),
            pltpu.VMEM((128 * _D,), jnp.float32),
            pltpu.SemaphoreType.DMA,
            pltpu.SemaphoreType.DMA,
            pltpu.SemaphoreType.DMA,
            pltpu.SemaphoreType.DMA,
        ],
        compiler_params=pltpu.CompilerParams(
            use_tc_tiling_on_sc=True,
            needs_layout_passes=False,
            disable_bounds_checks=True,
        ),
    )
    lin = k_a(table.T, tailp)

    k_b = pl.kernel(
        _body_b,
        out_type=jax.ShapeDtypeStruct((_L, 8, _B // 128, 8, 128), jnp.float32),
        mesh=mesh,
        scratch_types=[
            pltpu.VMEM((_PER_W,), jnp.int32),
            pltpu.VMEM((_CHUNK, _D), jnp.float32),
            pltpu.VMEM((_CHUNK, _D), jnp.float32),
            pltpu.VMEM((8, 8, _OTP), jnp.float32),
            pltpu.VMEM((8, 8, _OTP), jnp.float32),
            pltpu.SemaphoreType.DMA,
            pltpu.SemaphoreType.DMA,
            pltpu.SemaphoreType.DMA,
            pltpu.SemaphoreType.DMA,
        ],
        compiler_params=pltpu.CompilerParams(
            use_tc_tiling_on_sc=False,
            needs_layout_passes=False,
            disable_bounds_checks=True,
        ),
    )
    o5 = k_b(idx, lin.reshape(_V, _D))
    return o5.transpose((2, 4, 0, 1, 3)).reshape(_B, _L, _D)


# speedup vs baseline: 1.2734x; 1.2734x over previous
import jax
import jax.numpy as jnp
from jax import lax
from jax.experimental import pallas as pl
from jax.experimental.pallas import tpu as pltpu
from jax.experimental.pallas import tpu_sc as plsc

_B = 4096
_L = 200
_D = 64
_N = _B * _L
_NW = 32
_PER_W = _N // _NW
_CHUNK = 512
_CPW = _PER_W // _CHUNK
_JBLK = _CHUNK // 128
_CPL = _B // _CHUNK
_OTP = 129


def _body(
    idx_hbm, table_hbm, o5_hbm,
    idx_v, rows0, rows1, ot0, ot1,
    gsem0, gsem1, wsem0, wsem1,
):
    wid = lax.axis_index("s") * 2 + lax.axis_index("c")
    base_chunk = wid * _CPW

    pltpu.sync_copy(idx_hbm.at[pl.ds(wid * _PER_W, _PER_W)], idx_v)

    lane16 = lax.iota(jnp.int32, 16)
    tsg = []
    for g in range(_D // 16):
        d = g * 16 + lane16
        tsg.append((d >> 3, d & 7))

    rows = (rows0, rows1)
    gsems = (gsem0, gsem1)
    ots = (ot0, ot1)
    wsems = (wsem0, wsem1)

    def gather_copy(ci, p):
        return pltpu.make_async_copy(
            table_hbm.at[idx_v.at[pl.ds(ci * _CHUNK, _CHUNK)]],
            rows[p],
            gsems[p],
        )

    def write_copy(l, bj, q):
        return pltpu.make_async_copy(
            ots[q].at[:, :, pl.ds(0, 128)],
            o5_hbm.at[l, :, bj],
            wsems[q],
        )

    def process_chunk(ci, p):
        c = base_chunk + ci
        l = c // _CPL
        bblk0 = (c % _CPL) * _JBLK
        for j in range(_JBLK):
            q = j % 2
            m = ci * _JBLK + j

            @pl.when(m >= 2)
            def _wait_prev():
                write_copy(l, bblk0 + j, q).wait()

            @plsc.parallel_loop(0, 128, step=8)
            def _rowblk(rr0):
                for u in range(8):
                    rr = rr0 + u
                    lane_b = jnp.full((16,), rr, jnp.int32)
                    r = j * 128 + rr
                    for g in range(_D // 16):
                        vals = rows[p][r, pl.ds(g * 16, 16)]
                        plsc.store_scatter(
                            ots[q], [tsg[g][0], tsg[g][1], lane_b], vals
                        )
            write_copy(l, bblk0 + j, q).start()
        return l

    gather_copy(0, 0).start()

    def two_chunks(h, carry):
        for p in range(2):
            ci = 2 * h + p
            nci = ci + 1

            @pl.when(nci < _CPW)
            def _start_next():
                gather_copy(nci, (p + 1) % 2).start()

            gather_copy(ci, p).wait()
            process_chunk(ci, p)
        return carry

    lax.fori_loop(0, _CPW // 2, two_chunks, 0)

    write_copy(_L - 1, _B // 128 - 1, 0).wait()
    write_copy(_L - 1, _B // 128 - 1, 1).wait()


def kernel(x, table):
    idx = x.T.reshape(_N)
    mesh = plsc.VectorSubcoreMesh(core_axis_name="c", subcore_axis_name="s")
    k = pl.kernel(
        _body,
        out_type=jax.ShapeDtypeStruct((_L, 8, _B // 128, 8, 128), jnp.float32),
        mesh=mesh,
        scratch_types=[
            pltpu.VMEM((_PER_W,), jnp.int32),
            pltpu.VMEM((_CHUNK, _D), jnp.float32),
            pltpu.VMEM((_CHUNK, _D), jnp.float32),
            pltpu.VMEM((8, 8, _OTP), jnp.float32),
            pltpu.VMEM((8, 8, _OTP), jnp.float32),
            pltpu.SemaphoreType.DMA,
            pltpu.SemaphoreType.DMA,
            pltpu.SemaphoreType.DMA,
            pltpu.SemaphoreType.DMA,
        ],
        compiler_params=pltpu.CompilerParams(
            use_tc_tiling_on_sc=False,
            needs_layout_passes=False,
            disable_bounds_checks=True,
        ),
    )
    o5 = k(idx, table)
    return o5.transpose((2, 4, 0, 1, 3)).reshape(_B, _L, _D)
